# baseline (speedup 1.0000x reference)
import functools

import jax
import jax.numpy as jnp
from jax import lax
from jax.experimental import pallas as pl
from jax.experimental.pallas import tpu as pltpu
from jax.experimental.pallas import tpu_sc as plsc

NC, NS = 2, 16
NW = NC * NS


def _tc1_kernel(x_ref, w_ref, i12_ref):
    B = x_ref.shape[0]
    w = w_ref[...]
    K = w.shape[0]
    HW = x_ref.shape[2]
    T = B * HW

    xall = jnp.concatenate([x_ref[b] for b in range(B)], axis=1)
    wsq = jnp.sum(w * w, axis=1)
    dots = jax.lax.dot_general(
        w, xall, (((1,), (0,)), ((), ())),
        preferred_element_type=jnp.float32,
        precision=jax.lax.Precision.HIGHEST)
    score = wsq[:, None] - 2.0 * dots

    iota_k = jax.lax.broadcasted_iota(jnp.int32, (K, T), 0)
    big = jnp.int32(K)
    m1 = jnp.min(score, axis=0)
    i1 = jnp.min(jnp.where(score == m1[None, :], iota_k, big), axis=0)
    score2 = jnp.where(iota_k == i1[None, :], jnp.inf, score)
    m2 = jnp.min(score2, axis=0)
    i2 = jnp.min(jnp.where(score2 == m2[None, :], iota_k, big), axis=0)

    i12_ref[0, :T] = i1
    i12_ref[0, T:] = i2


def _sc_gather_body(w_hbm, idx_hbm, out_hbm, idx_v, rows_v, sem):
    n_per_w = idx_v.shape[0]
    wid = lax.axis_index("s") * NC + lax.axis_index("c")
    base = wid * n_per_w
    pltpu.sync_copy(idx_hbm.at[pl.ds(base, n_per_w)], idx_v)
    pltpu.async_copy(w_hbm.at[idx_v], rows_v, sem).wait()
    pltpu.sync_copy(rows_v, out_hbm.at[pl.ds(base, n_per_w)])


def _tc2_kernel(x_ref, cand_ref, i12_ref, code_ref, xnew_ref):
    B = x_ref.shape[0]
    HW = x_ref.shape[2]
    T = B * HW

    xall = jnp.concatenate([x_ref[b] for b in range(B)], axis=1)
    cc = jnp.transpose(cand_ref[...])
    c1 = cc[:, :T]
    c2 = cc[:, T:]
    i1 = i12_ref[0, :T]
    i2 = i12_ref[0, T:]

    df1 = xall - c1
    df2 = xall - c2
    d1 = jnp.sqrt(jnp.sum(df1 * df1, axis=0))
    d2 = jnp.sqrt(jnp.sum(df2 * df2, axis=0))

    take2 = (d2 < d1) | ((d2 == d1) & (i2 < i1))
    code_ref[0, :] = jnp.where(take2, i2, i1)
    xnew = jnp.where(take2[None, :], c2, c1)
    for b in range(B):
        xnew_ref[b] = xnew[:, b * HW:(b + 1) * HW]


def kernel(x, weight):
    B, C, H, W = x.shape
    HW = H * W
    K = weight.shape[0]
    T = B * HW
    xf = x.reshape(B, C, HW)

    i12 = pl.pallas_call(
        _tc1_kernel,
        out_shape=jax.ShapeDtypeStruct((1, 2 * T), jnp.int32),
    )(xf, weight)

    n_per_w = (2 * T) // NW
    mesh = plsc.VectorSubcoreMesh(core_axis_name="c", subcore_axis_name="s")
    sc_gather = functools.partial(
        pl.kernel,
        out_type=jax.ShapeDtypeStruct((2 * T, C), jnp.float32),
        mesh=mesh,
        scratch_types=[
            pltpu.VMEM((n_per_w,), jnp.int32),
            pltpu.VMEM((n_per_w, C), jnp.float32),
            pltpu.SemaphoreType.DMA,
        ],
    )(_sc_gather_body)
    cand = sc_gather(weight, i12.reshape(2 * T))

    code2, xnew = pl.pallas_call(
        _tc2_kernel,
        out_shape=[
            jax.ShapeDtypeStruct((1, T), jnp.int32),
            jax.ShapeDtypeStruct((B, C, HW), jnp.float32),
        ],
    )(xf, cand, i12)

    return code2.reshape(B, HW), xnew.reshape(B, C, H, W)


# speedup vs baseline: 2.4072x; 2.4072x over previous
import jax
import jax.numpy as jnp
from jax.experimental import pallas as pl


def _vq_kernel(x_ref, w_ref, code_ref, xnew_ref):
    B = x_ref.shape[0]
    w = w_ref[...]
    K = w.shape[0]
    HW = x_ref.shape[2]
    T = B * HW

    xall = jnp.concatenate([x_ref[b] for b in range(B)], axis=1)

    w_hi = w.astype(jnp.bfloat16)
    r1 = w - w_hi.astype(jnp.float32)
    w_mid = r1.astype(jnp.bfloat16)
    w_lo = (r1 - w_mid.astype(jnp.float32)).astype(jnp.bfloat16)

    x_hi = xall.astype(jnp.bfloat16)
    x_lo = (xall - x_hi.astype(jnp.float32)).astype(jnp.bfloat16)

    def dot_kt(wp, xp):
        return jax.lax.dot_general(
            wp, xp, (((1,), (0,)), ((), ())),
            preferred_element_type=jnp.float32)

    wsq = jnp.sum(w * w, axis=1)
    dots = (dot_kt(w_hi, x_hi) + dot_kt(w_hi, x_lo)) + dot_kt(w_mid, x_hi)
    score = wsq[:, None] - 2.0 * dots

    iota_k = jax.lax.broadcasted_iota(jnp.int32, (K, T), 0)

    i1 = jnp.argmin(score, axis=0).astype(jnp.int32)
    score2 = jnp.where(iota_k == i1[None, :], jnp.inf, score)
    i2 = jnp.argmin(score2, axis=0).astype(jnp.int32)

    ii = jnp.concatenate([i1, i2])
    oh = (jax.lax.broadcasted_iota(jnp.int32, (K, 2 * T), 0)
          == ii[None, :]).astype(jnp.bfloat16)

    def sel(part):
        return jax.lax.dot_general(
            part, oh, (((0,), (0,)), ((), ())),
            preferred_element_type=jnp.float32)

    cc = (sel(w_hi) + sel(w_mid)) + sel(w_lo)
    c1 = cc[:, :T]
    c2 = cc[:, T:]

    df1 = xall - c1
    df2 = xall - c2
    d1 = jnp.sqrt(jnp.sum(df1 * df1, axis=0))
    d2 = jnp.sqrt(jnp.sum(df2 * df2, axis=0))

    take2 = (d2 < d1) | ((d2 == d1) & (i2 < i1))
    code_ref[0, :] = jnp.where(take2, i2, i1)
    xnew = jnp.where(take2[None, :], c2, c1)
    for b in range(B):
        xnew_ref[b] = xnew[:, b * HW:(b + 1) * HW]


def kernel(x, weight):
    B, C, H, W = x.shape
    HW = H * W
    K = weight.shape[0]
    xf = x.reshape(B, C, HW)

    code2, xnew = pl.pallas_call(
        _vq_kernel,
        out_shape=[
            jax.ShapeDtypeStruct((1, B * HW), jnp.int32),
            jax.ShapeDtypeStruct((B, C, HW), jnp.float32),
        ],
    )(xf, weight)

    return code2.reshape(B, HW), xnew.reshape(B, C, H, W)


# speedup vs baseline: 2.4138x; 1.0028x over previous
import jax
import jax.numpy as jnp
from jax.experimental import pallas as pl


def _vq_kernel(x_ref, w_ref, code_ref, xnew_ref):
    B = x_ref.shape[0]
    w = w_ref[...]
    K = w.shape[0]
    HW = x_ref.shape[2]
    T = B * HW

    xall = jnp.concatenate([x_ref[b] for b in range(B)], axis=1)

    w_hi = w.astype(jnp.bfloat16)
    r1 = w - w_hi.astype(jnp.float32)
    w_mid = r1.astype(jnp.bfloat16)
    w_lo = (r1 - w_mid.astype(jnp.float32)).astype(jnp.bfloat16)

    x_hi = xall.astype(jnp.bfloat16)
    x_lo = (xall - x_hi.astype(jnp.float32)).astype(jnp.bfloat16)

    def dot_kt(wp, xp):
        return jax.lax.dot_general(
            wp, xp, (((1,), (0,)), ((), ())),
            preferred_element_type=jnp.float32)

    wsq = jnp.sum(w * w, axis=1)
    dots = (dot_kt(w_hi, x_hi) + dot_kt(w_hi, x_lo)) + dot_kt(w_mid, x_hi)
    score = wsq[:, None] - 2.0 * dots

    iota_k = jax.lax.broadcasted_iota(jnp.int32, (K, T), 0)

    i1 = jnp.argmin(score, axis=0).astype(jnp.int32)
    score2 = jnp.where(iota_k == i1[None, :], jnp.inf, score)
    i2 = jnp.argmin(score2, axis=0).astype(jnp.int32)

    ii = jnp.concatenate([i1, i2])
    oh = (jax.lax.broadcasted_iota(jnp.int32, (K, 2 * T), 0)
          == ii[None, :]).astype(jnp.bfloat16)

    def sel(part):
        return jax.lax.dot_general(
            part, oh, (((0,), (0,)), ((), ())),
            preferred_element_type=jnp.float32)

    cc = (sel(w_hi) + sel(w_mid)) + sel(w_lo)
    c1 = cc[:, :T]
    c2 = cc[:, T:]

    def tree_sum_sq(df):
        v = df * df
        C = v.shape[0]
        chunks = []
        for c0 in range(0, C, 128):
            acc = v[c0:c0 + 8]
            for j in range(1, 16):
                acc = acc + v[c0 + 8 * j:c0 + 8 * j + 8]
            a = acc[:4] + acc[4:]
            b = a[:2] + a[2:]
            chunks.append(b[0] + b[1])
        return chunks[0] + chunks[1]

    d1 = jnp.sqrt(tree_sum_sq(xall - c1))
    d2 = jnp.sqrt(tree_sum_sq(xall - c2))

    take2 = (d2 < d1) | ((d2 == d1) & (i2 < i1))
    code_ref[0, :] = jnp.where(take2, i2, i1)
    xnew = jnp.where(take2[None, :], c2, c1)
    for b in range(B):
        xnew_ref[b] = xnew[:, b * HW:(b + 1) * HW]


def kernel(x, weight):
    B, C, H, W = x.shape
    HW = H * W
    K = weight.shape[0]
    xf = x.reshape(B, C, HW)

    code2, xnew = pl.pallas_call(
        _vq_kernel,
        out_shape=[
            jax.ShapeDtypeStruct((1, B * HW), jnp.int32),
            jax.ShapeDtypeStruct((B, C, HW), jnp.float32),
        ],
    )(xf, weight)

    return code2.reshape(B, HW), xnew.reshape(B, C, H, W)
